# Initial kernel scaffold; baseline (speedup 1.0000x reference)
import jax
import jax.numpy as jnp
from jax.experimental import pallas as pl


def kernel(x, edge_index, W1, b1, W2, b2):
    raise NotImplementedError("write your pallas kernel here")


# speedup vs baseline: 72.8227x; 72.8227x over previous
import functools

import jax
import jax.numpy as jnp
from jax import lax
from jax.experimental import pallas as pl
from jax.experimental.pallas import tpu as pltpu
from jax.experimental.pallas import tpu_sc as plsc

NC = 2
NS = 16
NW = NC * NS
CH = 6400
LANES = 128
SUB = 8


def _edge_agg_kernel(ncols, NP, EPW):
    NPS = NP // NS
    nchunks = EPW // CH
    mesh = plsc.VectorSubcoreMesh(
        core_axis_name="c", subcore_axis_name="s", num_cores=NC, num_subcores=NS
    )

    scratch = [pltpu.VMEM((CH,), jnp.int32), pltpu.VMEM((CH,), jnp.int32)]
    scratch += [pltpu.VMEM((CH,), jnp.float32) for _ in range(ncols)]
    scratch += [pltpu.VMEM((NPS,), jnp.float32)]
    scratch += [pltpu.VMEM_SHARED((NP,), jnp.float32) for _ in range(ncols)]
    scratch += [pltpu.SemaphoreType.DMA for _ in range(ncols)]

    @functools.partial(
        pl.kernel,
        out_type=jax.ShapeDtypeStruct((NC, ncols, NP), jnp.float32),
        mesh=mesh,
        scratch_types=scratch,
    )
    def k(*refs):
        src_hbm, dst_hbm = refs[0], refs[1]
        tbls = refs[2:2 + ncols]
        zeros_hbm = refs[2 + ncols]
        out_hbm = refs[3 + ncols]
        sc = list(refs[4 + ncols:])
        sidx, didx = sc[0], sc[1]
        rows = sc[2:2 + ncols]
        zbuf = sc[2 + ncols]
        shared = sc[3 + ncols:3 + 2 * ncols]
        sems = sc[3 + 2 * ncols:3 + 3 * ncols]

        cid = lax.axis_index("c")
        sid = lax.axis_index("s")
        wid = cid * NS + sid

        base_n = sid * NPS
        pltpu.sync_copy(zeros_hbm, zbuf)
        for col in range(ncols):
            pltpu.sync_copy(zbuf, shared[col].at[pl.ds(base_n, NPS)])
        plsc.subcore_barrier()

        base_e = wid * EPW

        def body(kk, carry):
            off = base_e + kk * CH
            pltpu.sync_copy(src_hbm.at[pl.ds(off, CH)], sidx)
            pltpu.sync_copy(dst_hbm.at[pl.ds(off, CH)], didx)
            cps = [
                pltpu.async_copy(tbls[col].at[sidx], rows[col], sems[col])
                for col in range(ncols)
            ]
            for col in range(ncols):
                cps[col].wait()
                pltpu.sync_copy(rows[col], shared[col].at[didx], add=True)
            return carry

        lax.fori_loop(0, nchunks, body, 0)
        plsc.subcore_barrier()

        for col in range(ncols):
            pltpu.sync_copy(shared[col].at[pl.ds(base_n, NPS)], zbuf)
            pltpu.sync_copy(zbuf, out_hbm.at[cid, col, pl.ds(base_n, NPS)])

    return k


def _tc1_body(degp_ref, x_ref, dinv_ref, y0_ref, y1_ref):
    deg = degp_ref[0] + degp_ref[1] + 1.0
    dinv = lax.rsqrt(deg)
    dinv_ref[...] = dinv
    y0_ref[...] = x_ref[0] * dinv
    y1_ref[...] = x_ref[1] * dinv


def _tc2_body(H, accp_ref, x_ref, dinv_ref, w1_ref, b1_ref, w2_ref, y2_ref):
    dinv = dinv_ref[...]
    agg0 = dinv * (accp_ref[0, 0] + accp_ref[1, 0] + dinv * x_ref[0])
    agg1 = dinv * (accp_ref[0, 1] + accp_ref[1, 1] + dinv * x_ref[1])
    acc = jnp.zeros_like(dinv)
    for j in range(H):
        hj = jnp.maximum(agg0 * w1_ref[0, j] + agg1 * w1_ref[1, j] + b1_ref[j], 0.0)
        acc = acc + hj * w2_ref[j, 0]
    y2_ref[...] = dinv * acc


def _tc3_body(accp2_ref, dinv_ref, y2_ref, b2_ref, out_ref):
    dinv = dinv_ref[...]
    acc2 = accp2_ref[0, 0] + accp2_ref[1, 0]
    out_ref[...] = dinv * acc2 + dinv * y2_ref[...] + b2_ref[0]


def kernel(x, edge_index, W1, b1, W2, b2):
    N = x.shape[0]
    E = edge_index.shape[1]
    H = W1.shape[1]
    f32 = jnp.float32

    NP = ((N + 256) + 255) // 256 * 256
    NPR = NP // LANES
    NPS = NP // NS
    unit = NW * CH
    EP = (E + unit - 1) // unit * unit
    EPW = EP // NW

    pad_e = EP - E
    pad_idx = (jnp.arange(pad_e, dtype=edge_index.dtype) % 256) + N
    src = jnp.concatenate([edge_index[0], pad_idx])
    dst = jnp.concatenate([edge_index[1], pad_idx])

    xp = jnp.pad(x.T, ((0, 0), (0, NP - N)))
    zeros_v = jnp.zeros((NPS,), f32)
    ones_tbl = jnp.ones((NP,), f32)

    agg1k = _edge_agg_kernel(1, NP, EPW)
    degp = agg1k(dst, dst, ones_tbl, zeros_v)

    t1 = pl.pallas_call(
        _tc1_body,
        out_shape=[jax.ShapeDtypeStruct((NPR, LANES), f32)] * 3,
    )
    dinv, y0, y1 = t1(
        degp[:, 0].reshape(NC, NPR, LANES), xp.reshape(2, NPR, LANES)
    )

    agg2k = _edge_agg_kernel(2, NP, EPW)
    accp = agg2k(src, dst, y0.reshape(NP), y1.reshape(NP), zeros_v)

    smem = pl.BlockSpec(memory_space=pltpu.SMEM)
    t2 = pl.pallas_call(
        functools.partial(_tc2_body, H),
        out_shape=jax.ShapeDtypeStruct((NPR, LANES), f32),
        in_specs=[pl.BlockSpec(), pl.BlockSpec(), pl.BlockSpec(), smem, smem, smem],
    )
    y2 = t2(accp.reshape(NC, 2, NPR, LANES), xp.reshape(2, NPR, LANES), dinv,
            W1, b1, W2)

    accp2 = agg1k(src, dst, y2.reshape(NP), zeros_v)

    t3 = pl.pallas_call(
        _tc3_body,
        out_shape=jax.ShapeDtypeStruct((NPR, LANES), f32),
        in_specs=[pl.BlockSpec(), pl.BlockSpec(), pl.BlockSpec(), smem],
    )
    outg = t3(accp2.reshape(NC, 1, NPR, LANES), dinv, y2, b2)

    return outg.reshape(NP)[:N].reshape(N, 1)


# speedup vs baseline: 149.7621x; 2.0565x over previous
import functools

import jax
import jax.numpy as jnp
from jax import lax
from jax.experimental import pallas as pl
from jax.experimental.pallas import tpu as pltpu
from jax.experimental.pallas import tpu_sc as plsc

NC = 2
NS = 16
NW = NC * NS
LANES = 128


def _pick_chunks(epw):
    nchunks = max(1, -(-epw // 10240))
    while epw % nchunks != 0 or (epw // nchunks) % 8 != 0:
        nchunks += 1
    return nchunks


def _edge_agg_kernel(ncols, NP, EPW, gather):
    NPS = NP // NS
    nchunks = _pick_chunks(EPW)
    CH = EPW // nchunks
    mesh = plsc.VectorSubcoreMesh(
        core_axis_name="c", subcore_axis_name="s", num_cores=NC, num_subcores=NS
    )

    scratch = [pltpu.VMEM((CH,), jnp.int32) for _ in range(2)]
    scratch += [pltpu.VMEM((CH,), jnp.int32) for _ in range(2)]
    nrbuf = 2 if gather else 1
    scratch += [pltpu.VMEM((CH,), jnp.float32)
                for _ in range(ncols * nrbuf)]
    scratch += [pltpu.VMEM((NPS,), jnp.float32)]
    scratch += [pltpu.VMEM_SHARED((NP,), jnp.float32) for _ in range(ncols)]
    scratch += [pltpu.SemaphoreType.DMA for _ in range(2 * ncols)]

    @functools.partial(
        pl.kernel,
        out_type=jax.ShapeDtypeStruct((NC, ncols, NP), jnp.float32),
        mesh=mesh,
        scratch_types=scratch,
    )
    def k(*refs):
        ei_hbm = refs[0]
        ntbl = ncols if gather else 1
        tbls = refs[1:1 + ntbl]
        zeros_hbm = refs[1 + ntbl]
        out_hbm = refs[2 + ntbl]
        sc = list(refs[3 + ntbl:])
        sidx = sc[0:2]
        didx = sc[2:4]
        rows = [sc[4 + b * ncols: 4 + (b + 1) * ncols] for b in range(nrbuf)]
        p = 4 + ncols * nrbuf
        zbuf = sc[p]
        shared = sc[p + 1: p + 1 + ncols]
        semg = [sc[p + 1 + ncols + b * ncols:
                   p + 1 + ncols + (b + 1) * ncols] for b in range(2)]

        cid = lax.axis_index("c")
        sid = lax.axis_index("s")
        wid = cid * NS + sid

        base_n = sid * NPS
        pltpu.sync_copy(zeros_hbm, zbuf)
        for col in range(ncols):
            pltpu.sync_copy(zbuf, shared[col].at[pl.ds(base_n, NPS)])
        if not gather:
            pltpu.sync_copy(tbls[0], rows[0][0])
        plsc.subcore_barrier()

        base_e = wid * EPW

        EP = NW * EPW

        def load_idx(kk, b):
            off = base_e + kk * CH
            pltpu.sync_copy(ei_hbm.at[pl.ds(off, CH)], sidx[b])
            pltpu.sync_copy(ei_hbm.at[pl.ds(EP + off, CH)], didx[b])

        def start_gather(b):
            return [pltpu.async_copy(tbls[c].at[sidx[b]], rows[b][c], semg[b][c])
                    for c in range(ncols)]

        def do_scatter(b):
            rb = rows[b] if gather else rows[0]
            for c in range(ncols):
                pltpu.sync_copy(rb[c], shared[c].at[didx[b]], add=True)

        gd = [None, None]
        load_idx(0, 0)
        if gather:
            gd[0] = start_gather(0)
        for kk in range(nchunks):
            b = kk & 1
            nb = 1 - b
            if kk + 1 < nchunks:
                load_idx(kk + 1, nb)
                if gather:
                    gd[nb] = start_gather(nb)
            if gather:
                for d in gd[b]:
                    d.wait()
            do_scatter(b)
        plsc.subcore_barrier()

        for col in range(ncols):
            pltpu.sync_copy(shared[col].at[pl.ds(base_n, NPS)], zbuf)
            pltpu.sync_copy(zbuf, out_hbm.at[cid, col, pl.ds(base_n, NPS)])

    return k


def _round_bf16(a):
    u = lax.bitcast_convert_type(a, jnp.uint32)
    r = u + jnp.uint32(0x7FFF) + ((u >> 16) & jnp.uint32(1))
    return lax.bitcast_convert_type(r & jnp.uint32(0xFFFF0000), jnp.float32)


def _tc1_body(degp_ref, x_ref, dinv_ref, y0_ref, y1_ref):
    deg = degp_ref[0] + degp_ref[1] + 1.0
    dinv = 1.0 / jnp.sqrt(deg)
    dinv_ref[...] = dinv
    y0_ref[...] = x_ref[0] * dinv
    y1_ref[...] = x_ref[1] * dinv


def _tc2_body(H, accp_ref, x_ref, dinv_ref, w1_ref, b1_ref, w2_ref, y2_ref):
    dinv = dinv_ref[...]
    agg0 = dinv * (accp_ref[0, 0] + accp_ref[1, 0] + dinv * x_ref[0])
    agg1 = dinv * (accp_ref[0, 1] + accp_ref[1, 1] + dinv * x_ref[1])
    acc = jnp.zeros_like(dinv)
    for j in range(H):
        hj = jnp.maximum(agg0 * w1_ref[0, j] + agg1 * w1_ref[1, j] + b1_ref[j], 0.0)
        acc = acc + _round_bf16(hj) * w2_ref[j, 0]
    y2_ref[...] = dinv * acc


def _tc3_body(accp2_ref, dinv_ref, y2_ref, b2_ref, out_ref):
    dinv = dinv_ref[...]
    acc2 = accp2_ref[0, 0] + accp2_ref[1, 0]
    out_ref[...] = dinv * acc2 + dinv * y2_ref[...] + b2_ref[0]


def kernel(x, edge_index, W1, b1, W2, b2):
    N = x.shape[0]
    E = edge_index.shape[1]
    H = W1.shape[1]
    f32 = jnp.float32

    NP = ((N + 256) + 255) // 256 * 256
    NPR = NP // LANES
    NPS = NP // NS
    unit = NW * 8
    EP = (E + unit - 1) // unit * unit
    if EP != E:
        pad_idx = (jnp.arange(EP - E, dtype=edge_index.dtype) % 256) + N
        ei = jnp.concatenate(
            [edge_index, jnp.stack([pad_idx, pad_idx])], axis=1)
    else:
        ei = edge_index
    ei = ei.reshape(2 * EP)
    EPW = EP // NW

    xb = _round_bf16(x)
    xp = jnp.pad(xb.T, ((0, 0), (0, NP - N)))
    zeros_v = jnp.zeros((NPS,), f32)
    CH0 = EPW // _pick_chunks(EPW)
    ones_v = jnp.ones((CH0,), f32)

    degk = _edge_agg_kernel(1, NP, EPW, gather=False)
    degp = degk(ei, ones_v, zeros_v)

    t1 = pl.pallas_call(
        _tc1_body,
        out_shape=[jax.ShapeDtypeStruct((NPR, LANES), f32)] * 3,
    )
    dinv, y0, y1 = t1(
        degp[:, 0].reshape(NC, NPR, LANES), xp.reshape(2, NPR, LANES)
    )

    agg2k = _edge_agg_kernel(2, NP, EPW, gather=True)
    accp = agg2k(ei, y0.reshape(NP), y1.reshape(NP), zeros_v)

    smem = pl.BlockSpec(memory_space=pltpu.SMEM)
    t2 = pl.pallas_call(
        functools.partial(_tc2_body, H),
        out_shape=jax.ShapeDtypeStruct((NPR, LANES), f32),
        in_specs=[pl.BlockSpec(), pl.BlockSpec(), pl.BlockSpec(), smem, smem, smem],
    )
    W1b = _round_bf16(W1)
    W2b = _round_bf16(W2)
    y2 = t2(accp.reshape(NC, 2, NPR, LANES), xp.reshape(2, NPR, LANES), dinv,
            W1b, b1, W2b)

    agg1k = _edge_agg_kernel(1, NP, EPW, gather=True)
    accp2 = agg1k(ei, y2.reshape(NP), zeros_v)

    t3 = pl.pallas_call(
        _tc3_body,
        out_shape=jax.ShapeDtypeStruct((NPR, LANES), f32),
        in_specs=[pl.BlockSpec(), pl.BlockSpec(), pl.BlockSpec(), smem],
    )
    outg = t3(accp2.reshape(NC, 1, NPR, LANES), dinv, y2, b2)

    return outg.reshape(NP)[:N].reshape(N, 1)


# speedup vs baseline: 225.5237x; 1.5059x over previous
import functools

import jax
import jax.numpy as jnp
from jax import lax
from jax.experimental import pallas as pl
from jax.experimental.pallas import tpu as pltpu
from jax.experimental.pallas import tpu_sc as plsc

NC = 2
NS = 16
NW = NC * NS
LANES = 128


def _pick_chunks(epw):
    nchunks = max(1, -(-epw // 10240))
    while epw % nchunks != 0 or (epw // nchunks) % 8 != 0:
        nchunks += 1
    return nchunks


def _edge_agg_kernel(ncols, NP, EPW, gather):
    NPS = NP // NS
    nchunks = _pick_chunks(EPW)
    CH = EPW // nchunks
    mesh = plsc.VectorSubcoreMesh(
        core_axis_name="c", subcore_axis_name="s", num_cores=NC, num_subcores=NS
    )

    scratch = [pltpu.VMEM((CH,), jnp.int32) for _ in range(2)]
    scratch += [pltpu.VMEM((CH,), jnp.int32) for _ in range(2)]
    nrbuf = 2 if gather else 1
    scratch += [pltpu.VMEM((CH,), jnp.float32)
                for _ in range(ncols * nrbuf)]
    scratch += [pltpu.VMEM((NPS,), jnp.float32)]
    scratch += [pltpu.VMEM_SHARED((NP,), jnp.float32) for _ in range(ncols)]
    if gather:
        scratch += [pltpu.VMEM_SHARED((NP,), jnp.float32) for _ in range(ncols)]
    scratch += [pltpu.SemaphoreType.DMA for _ in range(2 * ncols)]

    @functools.partial(
        pl.kernel,
        out_type=jax.ShapeDtypeStruct((NC, ncols, NP), jnp.float32),
        mesh=mesh,
        scratch_types=scratch,
    )
    def k(*refs):
        ei_hbm = refs[0]
        ntbl = ncols if gather else 1
        tbls = refs[1:1 + ntbl]
        zeros_hbm = refs[1 + ntbl]
        out_hbm = refs[2 + ntbl]
        sc = list(refs[3 + ntbl:])
        sidx = sc[0:2]
        didx = sc[2:4]
        rows = [sc[4 + b * ncols: 4 + (b + 1) * ncols] for b in range(nrbuf)]
        p = 4 + ncols * nrbuf
        zbuf = sc[p]
        shared = sc[p + 1: p + 1 + ncols]
        q = p + 1 + ncols
        if gather:
            stbl = sc[q: q + ncols]
            q += ncols
        semg = [sc[q + b * ncols: q + (b + 1) * ncols] for b in range(2)]

        cid = lax.axis_index("c")
        sid = lax.axis_index("s")
        wid = cid * NS + sid

        base_n = sid * NPS
        pltpu.sync_copy(zeros_hbm, zbuf)
        for col in range(ncols):
            pltpu.sync_copy(zbuf, shared[col].at[pl.ds(base_n, NPS)])
        if gather:
            for col in range(ncols):
                pltpu.sync_copy(tbls[col].at[pl.ds(base_n, NPS)], zbuf)
                pltpu.sync_copy(zbuf, stbl[col].at[pl.ds(base_n, NPS)])
        if not gather:
            pltpu.sync_copy(tbls[0], rows[0][0])
        plsc.subcore_barrier()

        base_e = wid * EPW

        EP = NW * EPW

        def load_idx(kk, b):
            off = base_e + kk * CH
            pltpu.sync_copy(ei_hbm.at[pl.ds(off, CH)], sidx[b])
            pltpu.sync_copy(ei_hbm.at[pl.ds(EP + off, CH)], didx[b])

        def start_gather(b):
            return [pltpu.async_copy(stbl[c].at[sidx[b]], rows[b][c], semg[b][c])
                    for c in range(ncols)]

        def do_scatter(b):
            rb = rows[b] if gather else rows[0]
            for c in range(ncols):
                pltpu.sync_copy(rb[c], shared[c].at[didx[b]], add=True)

        gd = [None, None]
        load_idx(0, 0)
        if gather:
            gd[0] = start_gather(0)
        for kk in range(nchunks):
            b = kk & 1
            nb = 1 - b
            if kk + 1 < nchunks:
                load_idx(kk + 1, nb)
                if gather:
                    gd[nb] = start_gather(nb)
            if gather:
                for d in gd[b]:
                    d.wait()
            do_scatter(b)
        plsc.subcore_barrier()

        for col in range(ncols):
            pltpu.sync_copy(shared[col].at[pl.ds(base_n, NPS)], zbuf)
            pltpu.sync_copy(zbuf, out_hbm.at[cid, col, pl.ds(base_n, NPS)])

    return k


def _round_bf16(a):
    u = lax.bitcast_convert_type(a, jnp.uint32)
    r = u + jnp.uint32(0x7FFF) + ((u >> 16) & jnp.uint32(1))
    return lax.bitcast_convert_type(r & jnp.uint32(0xFFFF0000), jnp.float32)


def _tc1_body(degp_ref, x_ref, dinv_ref, y0_ref, y1_ref):
    deg = degp_ref[0] + degp_ref[1] + 1.0
    dinv = 1.0 / jnp.sqrt(deg)
    dinv_ref[...] = dinv
    y0_ref[...] = x_ref[0] * dinv
    y1_ref[...] = x_ref[1] * dinv


def _tc2_body(H, accp_ref, x_ref, dinv_ref, w1_ref, b1_ref, w2_ref, y2_ref):
    dinv = dinv_ref[...]
    agg0 = dinv * (accp_ref[0, 0] + accp_ref[1, 0] + dinv * x_ref[0])
    agg1 = dinv * (accp_ref[0, 1] + accp_ref[1, 1] + dinv * x_ref[1])
    acc = jnp.zeros_like(dinv)
    for j in range(H):
        hj = jnp.maximum(agg0 * w1_ref[0, j] + agg1 * w1_ref[1, j] + b1_ref[j], 0.0)
        acc = acc + _round_bf16(hj) * w2_ref[j, 0]
    y2_ref[...] = dinv * acc


def _tc3_body(accp2_ref, dinv_ref, y2_ref, b2_ref, out_ref):
    dinv = dinv_ref[...]
    acc2 = accp2_ref[0, 0] + accp2_ref[1, 0]
    out_ref[...] = dinv * acc2 + dinv * y2_ref[...] + b2_ref[0]


def kernel(x, edge_index, W1, b1, W2, b2):
    N = x.shape[0]
    E = edge_index.shape[1]
    H = W1.shape[1]
    f32 = jnp.float32

    NP = ((N + 256) + 255) // 256 * 256
    NPR = NP // LANES
    NPS = NP // NS
    unit = NW * 8
    EP = (E + unit - 1) // unit * unit
    if EP != E:
        pad_idx = (jnp.arange(EP - E, dtype=edge_index.dtype) % 256) + N
        ei = jnp.concatenate(
            [edge_index, jnp.stack([pad_idx, pad_idx])], axis=1)
    else:
        ei = edge_index
    ei = ei.reshape(2 * EP)
    EPW = EP // NW

    xb = _round_bf16(x)
    xp = jnp.pad(xb.T, ((0, 0), (0, NP - N)))
    zeros_v = jnp.zeros((NPS,), f32)
    CH0 = EPW // _pick_chunks(EPW)
    ones_v = jnp.ones((CH0,), f32)

    degk = _edge_agg_kernel(1, NP, EPW, gather=False)
    degp = degk(ei, ones_v, zeros_v)

    t1 = pl.pallas_call(
        _tc1_body,
        out_shape=[jax.ShapeDtypeStruct((NPR, LANES), f32)] * 3,
    )
    dinv, y0, y1 = t1(
        degp[:, 0].reshape(NC, NPR, LANES), xp.reshape(2, NPR, LANES)
    )

    agg2k = _edge_agg_kernel(2, NP, EPW, gather=True)
    accp = agg2k(ei, y0.reshape(NP), y1.reshape(NP), zeros_v)

    smem = pl.BlockSpec(memory_space=pltpu.SMEM)
    t2 = pl.pallas_call(
        functools.partial(_tc2_body, H),
        out_shape=jax.ShapeDtypeStruct((NPR, LANES), f32),
        in_specs=[pl.BlockSpec(), pl.BlockSpec(), pl.BlockSpec(), smem, smem, smem],
    )
    W1b = _round_bf16(W1)
    W2b = _round_bf16(W2)
    y2 = t2(accp.reshape(NC, 2, NPR, LANES), xp.reshape(2, NPR, LANES), dinv,
            W1b, b1, W2b)

    agg1k = _edge_agg_kernel(1, NP, EPW, gather=True)
    accp2 = agg1k(ei, y2.reshape(NP), zeros_v)

    t3 = pl.pallas_call(
        _tc3_body,
        out_shape=jax.ShapeDtypeStruct((NPR, LANES), f32),
        in_specs=[pl.BlockSpec(), pl.BlockSpec(), pl.BlockSpec(), smem],
    )
    outg = t3(accp2.reshape(NC, 1, NPR, LANES), dinv, y2, b2)

    return outg.reshape(NP)[:N].reshape(N, 1)
